# baseline (speedup 1.0000x reference)
import functools

import jax
import jax.numpy as jnp
from jax import lax
from jax.experimental import pallas as pl
from jax.experimental.pallas import tpu as pltpu
from jax.experimental.pallas import tpu_sc as plsc

N = 100000
E = 3200000
NP = 102400
NC = 2
NS = 16
L = 16
NW = NC * NS
CB = 800
NCH = 125
EW = NCH * CB
EP = NW * EW
TSL = NP // NS
PBLK = 640
UBLK = 320

_mesh = plsc.VectorSubcoreMesh(
    core_axis_name="c", subcore_axis_name="s", num_cores=NC, num_subcores=NS)


def _iota16():
    return jnp.arange(L, dtype=jnp.int32)


def _bf16hi(v):
    b = plsc.bitcast(v, jnp.int32)
    r = (b + 0x7FFF + ((b >> 16) & 1)) & (-65536)
    return plsc.bitcast(r, jnp.float32)


def _em(a, w, w_hi):
    ahi = _bf16hi(a)
    return ahi * w + (a - ahi) * w_hi


def _bcast(ref, row):
    return ref[row]


def _edge_pass(first: bool):
    PB = 320 if first else PBLK
    outs = [
        jax.ShapeDtypeStruct((NC * NP,), jnp.float32),
        jax.ShapeDtypeStruct((4 * NP,), jnp.float32),
        jax.ShapeDtypeStruct((NC * NP,), jnp.float32),
    ]
    if first:
        outs.append(jax.ShapeDtypeStruct((NC * NP,), jnp.float32))

    scratch = [
        pltpu.VMEM((N,), jnp.float32),
        pltpu.VMEM((PB,), jnp.float32),
        pltpu.VMEM((PB,), jnp.float32),
        pltpu.VMEM((PB,), jnp.float32),
        pltpu.VMEM((PB,), jnp.float32),
        pltpu.VMEM((PB,), jnp.float32),
        pltpu.VMEM((PB,), jnp.float32),
        pltpu.VMEM((CB,), jnp.int32),
        pltpu.VMEM((CB,), jnp.int32),
        pltpu.VMEM((CB,), jnp.float32),
        pltpu.VMEM((CB,), jnp.float32),
        pltpu.VMEM((CB,), jnp.float32),
        pltpu.VMEM((CB,), jnp.float32),
        pltpu.VMEM((CB,), jnp.int32),
        pltpu.VMEM((CB,), jnp.int32),
        pltpu.VMEM((CB,), jnp.float32),
        pltpu.VMEM((CB,), jnp.float32),
        pltpu.VMEM((CB,), jnp.float32),
        pltpu.VMEM((CB,), jnp.float32),
        pltpu.VMEM((CB,), jnp.float32),
        pltpu.VMEM((24, L), jnp.float32),
        pltpu.VMEM_SHARED((NP,), jnp.float32),
        pltpu.SemaphoreType.DMA,
        pltpu.SemaphoreType.DMA,
        pltpu.SemaphoreType.DMA,
    ]
    if first:
        scratch.append(pltpu.VMEM_SHARED((NP,), jnp.float32))
    if not first:
        scratch.extend([
            pltpu.VMEM((PB,), jnp.float32),
            pltpu.VMEM((PB,), jnp.float32),
            pltpu.VMEM((PB,), jnp.float32),
            pltpu.VMEM((PB,), jnp.float32),
        ])

    def body(*refs):
        if first:
            (zT, srcr, dstr, rr0, rr1, rr2, rhr, pp,
             t_out, x_out, s_buf, deg_out,
             s_v, z0, z1, z2, z3, aob, sob,
             src_a, dst_a, r0_a, r1_a, r2_a, rh_a, src_b, dst_b, r0_b, r1_b, r2_b, rh_b,
             mbuf, pv, t_sh, sem_a, sem_b, sem_sc, deg_sh) = refs
        else:
            (x_in, t_in, deg_in, srcr, dstr, rr0, rr1, rr2, rhr, pp,
             t_out, x_out, s_buf,
             s_v, z0, z1, z2, z3, aob, sob,
             src_a, dst_a, r0_a, r1_a, r2_a, rh_a, src_b, dst_b, r0_b, r1_b, r2_b, rh_b,
             mbuf, pv, t_sh, sem_a, sem_b, sem_sc,
             t0b, t1b, g0b, g1b) = refs

        c = lax.axis_index("c")
        s = lax.axis_index("s")
        wid = c * NS + s
        it = _iota16()

        pltpu.sync_copy(pp, pv)

        @pl.loop(0, PB // L)
        def _(i):
            aob[pl.ds(i * L, L)] = jnp.zeros((L,), jnp.float32)

        base_n = s * TSL
        for blk in range(TSL // PB):
            off = base_n + blk * PB
            pltpu.sync_copy(aob, t_sh.at[pl.ds(off, PB)])
            if first:
                pltpu.sync_copy(aob, deg_sh.at[pl.ds(off, PB)])

        ws0, ws1, ws2, ws3 = (pv[5], pv[6], pv[7], pv[8])
        wsh0, wsh1, wsh2, wsh3 = (pv[16], pv[17], pv[18], pv[19])
        if not first:
            wd0, wd1, wd2, wd3 = (pv[10], pv[11], pv[12], pv[13])
            wdh0, wdh1, wdh2, wdh3 = (pv[20], pv[21], pv[22], pv[23])

        @pl.loop(0, TSL // PB)
        def _(blk):
            off = base_n + blk * PB
            xsrc = zT if first else x_in
            loads = [(xsrc.at[pl.ds(off, PB)], z0),
                     (xsrc.at[pl.ds(NP + off, PB)], z1),
                     (xsrc.at[pl.ds(2 * NP + off, PB)], z2),
                     (xsrc.at[pl.ds(3 * NP + off, PB)], z3)]
            if not first:
                loads += [(t_in.at[pl.ds(off, PB)], t0b),
                          (t_in.at[pl.ds(NP + off, PB)], t1b),
                          (deg_in.at[pl.ds(off, PB)], g0b),
                          (deg_in.at[pl.ds(NP + off, PB)], g1b)]
            for sref, dref in loads:
                pltpu.async_copy(sref, dref, sem_a)
            for sref, dref in loads:
                pltpu.make_async_copy(sref, dref, sem_a).wait()

            @pl.loop(0, PB // L)
            def _(g):
                o = g * L
                zv0 = z0[pl.ds(o, L)]
                zv1 = z1[pl.ds(o, L)]
                zv2 = z2[pl.ds(o, L)]
                zv3 = z3[pl.ds(o, L)]
                if not first:
                    dv = (_em(zv0, wd0, wdh0) + _em(zv1, wd1, wdh1)
                          + _em(zv2, wd2, wdh2) + _em(zv3, wd3, wdh3))
                    agg = (t0b[pl.ds(o, L)] + t1b[pl.ds(o, L)]
                           + (g0b[pl.ds(o, L)] + g1b[pl.ds(o, L)]) * dv)
                    zv0 = zv0 + agg
                    zv1 = zv1 + agg
                    zv2 = zv2 + agg
                    zv3 = zv3 + agg
                    z0[pl.ds(o, L)] = zv0
                    z1[pl.ds(o, L)] = zv1
                    z2[pl.ds(o, L)] = zv2
                    z3[pl.ds(o, L)] = zv3
                sob[pl.ds(o, L)] = (
                    _em(zv0, ws0, wsh0) + _em(zv1, ws1, wsh1)
                    + _em(zv2, ws2, wsh2) + _em(zv3, ws3, wsh3))
            if not first:
                @pl.when(c == 0)
                def _():
                    pltpu.sync_copy(z0, x_out.at[pl.ds(off, PB)])
                    pltpu.sync_copy(z1, x_out.at[pl.ds(NP + off, PB)])
                    pltpu.sync_copy(z2, x_out.at[pl.ds(2 * NP + off, PB)])
                    pltpu.sync_copy(z3, x_out.at[pl.ds(3 * NP + off, PB)])
            pltpu.sync_copy(sob, s_buf.at[pl.ds(c * NP + off, PB)])

        plsc.subcore_barrier()
        pltpu.sync_copy(s_buf.at[pl.ds(c * NP, N)], s_v)

        p0, p1, p2, p3, p4 = pv[0], pv[1], pv[2], pv[3], pv[4]

        def in_slices(ck):
            base = wid * EW + ck * CB
            return (srcr.at[pl.ds(base, CB)],
                    dstr.at[pl.ds(base, CB)],
                    rr0.at[pl.ds(base, CB)],
                    rr1.at[pl.ds(base, CB)],
                    rr2.at[pl.ds(base, CB)],
                    rhr.at[pl.ds(base, CB)])

        def issue_loads(ck, bufs, sem):
            for sref, dref in zip(in_slices(ck), bufs):
                pltpu.async_copy(sref, dref, sem)

        def wait_loads(ck, bufs, sem):
            for sref, dref in zip(in_slices(ck), bufs):
                pltpu.make_async_copy(sref, dref, sem).wait()

        def drain_sc(nds):
            for _ in range(nds):
                pltpu.make_async_copy(mbuf, t_sh.at[dst_a], sem_sc).wait()

        def do_chunk(bufs):
            src_v, dst_v, r0_v, r1_v, r2_v, rh_v = bufs

            @pl.loop(0, CB // L, unroll=8)
            def _(g):
                b = g * L
                sidx = src_v[pl.ds(b, L)]
                sval = plsc.load_gather(s_v, [sidx])
                r0 = r0_v[pl.ds(b, L)]
                r1 = r1_v[pl.ds(b, L)]
                r2 = r2_v[pl.ds(b, L)]
                rh = rh_v[pl.ds(b, L)]
                mbuf[pl.ds(b, L)] = (
                    sval + r0 * p0 + r1 * p1 + r2 * p2 + rh * p3 + p4)
            pltpu.async_copy(mbuf, t_sh.at[dst_v], sem_sc, add=True)
            drain_sc(1)
            if first:
                ones = jnp.ones((L,), jnp.float32)

                @pl.loop(0, CB // L)
                def _(g):
                    mbuf[pl.ds(g * L, L)] = ones
                pltpu.async_copy(mbuf, deg_sh.at[dst_v], sem_sc, add=True)
                drain_sc(1)

        bufs_a = (src_a, dst_a, r0_a, r1_a, r2_a, rh_a)
        bufs_b = (src_b, dst_b, r0_b, r1_b, r2_b, rh_b)

        issue_loads(0, bufs_a, sem_a)

        @pl.loop(0, NCH // 2)
        def _(half):
            k = half * 2
            issue_loads(k + 1, bufs_b, sem_b)
            wait_loads(k, bufs_a, sem_a)
            do_chunk(bufs_a)
            issue_loads(k + 2, bufs_a, sem_a)
            wait_loads(k + 1, bufs_b, sem_b)
            do_chunk(bufs_b)

        wait_loads(NCH - 1, bufs_a, sem_a)
        do_chunk(bufs_a)
        plsc.subcore_barrier()

        sl = pl.ds(base_n, TSL)
        osl = pl.ds(c * NP + base_n, TSL)
        pltpu.sync_copy(t_sh.at[sl], t_out.at[osl])

        if first:
            pltpu.sync_copy(deg_sh.at[sl], deg_out.at[osl])

    return pl.kernel(body, out_type=tuple(outs), mesh=_mesh,
                     scratch_types=scratch,
                     compiler_params=pltpu.CompilerParams(
                         needs_layout_passes=False))


def _finalize():
    scratch = [
        pltpu.VMEM((4 * UBLK,), jnp.float32),
        pltpu.VMEM((UBLK,), jnp.float32),
        pltpu.VMEM((UBLK,), jnp.float32),
        pltpu.VMEM((UBLK,), jnp.float32),
        pltpu.VMEM((UBLK,), jnp.float32),
        pltpu.VMEM((4 * UBLK,), jnp.float32),
        pltpu.VMEM((24, L), jnp.float32),
    ]

    def body(x_in, t_in, deg_in, pp, x_out,
             cb, t0b, t1b, g0b, g1b, xb, pv):
        c = lax.axis_index("c")
        s = lax.axis_index("s")
        wid = c * NS + s
        it = _iota16()
        pltpu.sync_copy(pp, pv)
        wd0, wd1, wd2, wd3 = (pv[10], pv[11], pv[12], pv[13])
        wdh0, wdh1, wdh2, wdh3 = (pv[20], pv[21], pv[22], pv[23])
        base_n = wid * (NP // NW)

        @pl.loop(0, NP // NW // UBLK)
        def _(blk):
            off = base_n + blk * UBLK
            loads = [(x_in.at[pl.ds(off, UBLK)], cb.at[pl.ds(0, UBLK)]),
                     (x_in.at[pl.ds(NP + off, UBLK)],
                      cb.at[pl.ds(UBLK, UBLK)]),
                     (x_in.at[pl.ds(2 * NP + off, UBLK)],
                      cb.at[pl.ds(2 * UBLK, UBLK)]),
                     (x_in.at[pl.ds(3 * NP + off, UBLK)],
                      cb.at[pl.ds(3 * UBLK, UBLK)]),
                     (t_in.at[pl.ds(off, UBLK)], t0b),
                     (t_in.at[pl.ds(NP + off, UBLK)], t1b),
                     (deg_in.at[pl.ds(off, UBLK)], g0b),
                     (deg_in.at[pl.ds(NP + off, UBLK)], g1b)]
            for sref, dref in loads:
                pltpu.sync_copy(sref, dref)

            @pl.loop(0, UBLK // L)
            def _(g):
                o = g * L
                xv0 = cb[pl.ds(o, L)]
                xv1 = cb[pl.ds(UBLK + o, L)]
                xv2 = cb[pl.ds(2 * UBLK + o, L)]
                xv3 = cb[pl.ds(3 * UBLK + o, L)]
                dv = (_em(xv0, wd0, wdh0) + _em(xv1, wd1, wdh1)
                      + _em(xv2, wd2, wdh2) + _em(xv3, wd3, wdh3))
                agg = (t0b[pl.ds(o, L)] + t1b[pl.ds(o, L)]
                       + (g0b[pl.ds(o, L)] + g1b[pl.ds(o, L)]) * dv)
                cb[pl.ds(o, L)] = xv0 + agg
                cb[pl.ds(UBLK + o, L)] = xv1 + agg
                cb[pl.ds(2 * UBLK + o, L)] = xv2 + agg
                cb[pl.ds(3 * UBLK + o, L)] = xv3 + agg

            @pl.loop(0, UBLK // L)
            def _(g):
                for q in range(4):
                    p = g * 64 + q * L
                    ai = (it & 3) * UBLK + g * L + q * 4 + (it >> 2)
                    xb[pl.ds(p, L)] = plsc.load_gather(cb, [ai])
            pltpu.sync_copy(xb, x_out.at[pl.ds(4 * off, 4 * UBLK)])

    return pl.kernel(
        body, out_type=jax.ShapeDtypeStruct((NP * 4,), jnp.float32),
        mesh=_mesh, scratch_types=scratch,
        compiler_params=pltpu.CompilerParams(needs_layout_passes=False))


def kernel(z, src, dst, r, r_hat, W, b):
    f32 = jnp.float32
    srcp = src.astype(jnp.int32)
    dstp = dst.astype(jnp.int32)
    rc0 = r[:, 0].astype(f32)
    rc1 = r[:, 1].astype(f32)
    rc2 = r[:, 2].astype(f32)
    rhf = r_hat[:, 0].astype(f32)
    zT = jnp.zeros((4, NP), f32).at[:, :N].set(z.astype(f32).T).reshape(-1)

    Wf = W.astype(f32)
    bf = b.astype(f32)

    def params_for(i, iprev):
        rows = [Wf[i, 8, 0], Wf[i, 9, 0], Wf[i, 10, 0], Wf[i, 11, 0],
                bf[i, 0],
                Wf[i, 0, 0], Wf[i, 1, 0], Wf[i, 2, 0], Wf[i, 3, 0],
                jnp.sum(Wf[i, 0:4, 0])]
        if iprev is None:
            rows += [jnp.zeros(()), jnp.zeros(()), jnp.zeros(()),
                     jnp.zeros(()), jnp.zeros(())]
        else:
            rows += [Wf[iprev, 4, 0], Wf[iprev, 5, 0], Wf[iprev, 6, 0],
                     Wf[iprev, 7, 0], jnp.sum(Wf[iprev, 4:8, 0])]
        def hi(v):
            return v.astype(jnp.bfloat16).astype(f32)
        rows += [jnp.zeros(())]
        rows += [hi(Wf[i, 0, 0]), hi(Wf[i, 1, 0]), hi(Wf[i, 2, 0]),
                 hi(Wf[i, 3, 0])]
        if iprev is None:
            rows += [jnp.zeros(()), jnp.zeros(()), jnp.zeros(()),
                     jnp.zeros(())]
        else:
            rows += [hi(Wf[iprev, 4, 0]), hi(Wf[iprev, 5, 0]),
                     hi(Wf[iprev, 6, 0]), hi(Wf[iprev, 7, 0])]
        return jnp.broadcast_to(jnp.stack(rows)[:, None], (24, L))

    e_first = _edge_pass(True)
    e_next = _edge_pass(False)
    fin = _finalize()

    t0p, _x0, _sb0, degp = e_first(zT, srcp, dstp, rc0, rc1, rc2, rhf,
                                   params_for(0, None))
    t1p, x1, _sb1 = e_next(zT, t0p, degp, srcp, dstp, rc0, rc1, rc2,
                           rhf, params_for(1, 0))
    t2p, x2, _sb2 = e_next(x1, t1p, degp, srcp, dstp, rc0, rc1, rc2,
                           rhf, params_for(2, 1))
    xf = fin(x2, t2p, degp, params_for(0, 2))
    return xf.reshape(NP, 4)[:N]


# speedup vs baseline: 1.0769x; 1.0769x over previous
import functools

import jax
import jax.numpy as jnp
from jax import lax
from jax.experimental import pallas as pl
from jax.experimental.pallas import tpu as pltpu
from jax.experimental.pallas import tpu_sc as plsc

N = 100000
E = 3200000
NP = 102400
NC = 2
NS = 16
L = 16
NW = NC * NS
CB = 1024
NCH = 98
EW = NCH * CB
EP = NW * EW
TSL = NP // NS
PBLK = 640
UBLK = 320

_mesh = plsc.VectorSubcoreMesh(
    core_axis_name="c", subcore_axis_name="s", num_cores=NC, num_subcores=NS)


def _iota16():
    return jnp.arange(L, dtype=jnp.int32)


def _bcast(ref, row):
    return ref[row]


def _edge_pass(first: bool):
    PB = 320 if first else PBLK
    outs = [
        jax.ShapeDtypeStruct((NC * NP,), jnp.float32),
        jax.ShapeDtypeStruct((NP,), jnp.float32),
        jax.ShapeDtypeStruct((NC * NP,), jnp.float32),
    ]
    if first:
        outs.append(jax.ShapeDtypeStruct((NC * NP,), jnp.float32))

    scratch = [
        pltpu.VMEM((N,), jnp.float32),
        pltpu.VMEM((PB,), jnp.float32),
        pltpu.VMEM((PB,), jnp.float32),
        pltpu.VMEM((PB,), jnp.float32),
        pltpu.VMEM((PB,), jnp.float32),
        pltpu.VMEM((PB,), jnp.float32),
        pltpu.VMEM((PB,), jnp.float32),
        pltpu.VMEM((CB,), jnp.int32),
        pltpu.VMEM((CB,), jnp.int32),
        pltpu.VMEM((CB,), jnp.float32),
        pltpu.VMEM((CB,), jnp.float32),
        pltpu.VMEM((CB,), jnp.float32),
        pltpu.VMEM((CB,), jnp.float32),
        pltpu.VMEM((CB,), jnp.int32),
        pltpu.VMEM((CB,), jnp.int32),
        pltpu.VMEM((CB,), jnp.float32),
        pltpu.VMEM((CB,), jnp.float32),
        pltpu.VMEM((CB,), jnp.float32),
        pltpu.VMEM((CB,), jnp.float32),
        pltpu.VMEM((CB,), jnp.float32),
        pltpu.VMEM((15, L), jnp.float32),
        pltpu.VMEM_SHARED((NP,), jnp.float32),
        pltpu.SemaphoreType.DMA,
        pltpu.SemaphoreType.DMA,
        pltpu.SemaphoreType.DMA,
    ]
    if first:
        scratch.append(pltpu.VMEM_SHARED((NP,), jnp.float32))
    if not first:
        scratch.extend([
            pltpu.VMEM((PB,), jnp.float32),
            pltpu.VMEM((PB,), jnp.float32),
            pltpu.VMEM((PB,), jnp.float32),
            pltpu.VMEM((PB,), jnp.float32),
            pltpu.VMEM((PB,), jnp.float32),
        ])

    def body(*refs):
        if first:
            (zT, srcr, dstr, rr0, rr1, rr2, rhr, pp,
             t_out, a_out, s_buf, deg_out,
             s_v, z0, z1, z2, z3, aob, sob,
             src_a, dst_a, r0_a, r1_a, r2_a, rh_a, src_b, dst_b, r0_b, r1_b, r2_b, rh_b,
             mbuf, pv, t_sh, sem_a, sem_b, sem_sc, deg_sh) = refs
        else:
            (zT, a_in, t_in, deg_in, srcr, dstr, rr0, rr1, rr2, rhr, pp,
             t_out, a_out, s_buf,
             s_v, z0, z1, z2, z3, aob, sob,
             src_a, dst_a, r0_a, r1_a, r2_a, rh_a, src_b, dst_b, r0_b, r1_b, r2_b, rh_b,
             mbuf, pv, t_sh, sem_a, sem_b, sem_sc,
             ab, t0b, t1b, g0b, g1b) = refs

        c = lax.axis_index("c")
        s = lax.axis_index("s")
        wid = c * NS + s
        it = _iota16()

        pltpu.sync_copy(pp, pv)

        @pl.loop(0, PB // L)
        def _(i):
            aob[pl.ds(i * L, L)] = jnp.zeros((L,), jnp.float32)

        base_n = s * TSL
        for blk in range(TSL // PB):
            off = base_n + blk * PB
            pltpu.sync_copy(aob, t_sh.at[pl.ds(off, PB)])
            if first:
                pltpu.sync_copy(aob, deg_sh.at[pl.ds(off, PB)])

                @pl.when(c == 0)
                def _(off=off):
                    pltpu.sync_copy(aob, a_out.at[pl.ds(off, PB)])

        ws0, ws1, ws2, ws3, ssum = (pv[5], pv[6], pv[7], pv[8], pv[9])
        if not first:
            wd0, wd1, wd2, wd3, sdum = (pv[10], pv[11], pv[12], pv[13], pv[14])

        @pl.loop(0, TSL // PB)
        def _(blk):
            off = base_n + blk * PB
            loads = [(zT.at[pl.ds(off, PB)], z0),
                     (zT.at[pl.ds(NP + off, PB)], z1),
                     (zT.at[pl.ds(2 * NP + off, PB)], z2),
                     (zT.at[pl.ds(3 * NP + off, PB)], z3)]
            if not first:
                loads += [(a_in.at[pl.ds(off, PB)], ab),
                          (t_in.at[pl.ds(off, PB)], t0b),
                          (t_in.at[pl.ds(NP + off, PB)], t1b),
                          (deg_in.at[pl.ds(off, PB)], g0b),
                          (deg_in.at[pl.ds(NP + off, PB)], g1b)]
            for sref, dref in loads:
                pltpu.async_copy(sref, dref, sem_a)
            for sref, dref in loads:
                pltpu.make_async_copy(sref, dref, sem_a).wait()

            @pl.loop(0, PB // L)
            def _(g):
                o = g * L
                zv0 = z0[pl.ds(o, L)]
                zv1 = z1[pl.ds(o, L)]
                zv2 = z2[pl.ds(o, L)]
                zv3 = z3[pl.ds(o, L)]
                if first:
                    anew = jnp.zeros((L,), jnp.float32)
                else:
                    av = ab[pl.ds(o, L)]
                    dv = (zv0 * wd0 + zv1 * wd1 + zv2 * wd2 + zv3 * wd3
                          + sdum * av)
                    anew = (av + t0b[pl.ds(o, L)] + t1b[pl.ds(o, L)]
                            + (g0b[pl.ds(o, L)] + g1b[pl.ds(o, L)]) * dv)
                    aob[pl.ds(o, L)] = anew
                sob[pl.ds(o, L)] = (zv0 * ws0 + zv1 * ws1 + zv2 * ws2
                                    + zv3 * ws3 + ssum * anew)
            if not first:
                @pl.when(c == 0)
                def _():
                    pltpu.sync_copy(aob, a_out.at[pl.ds(off, PB)])
            pltpu.sync_copy(sob, s_buf.at[pl.ds(c * NP + off, PB)])

        plsc.subcore_barrier()
        pltpu.sync_copy(s_buf.at[pl.ds(c * NP, N)], s_v)

        p0, p1, p2, p3, p4 = pv[0], pv[1], pv[2], pv[3], pv[4]

        def in_slices(ck):
            base = wid * EW + ck * CB
            base_r = jnp.minimum(base, E - CB)
            return (srcr.at[pl.ds(base, CB)],
                    dstr.at[pl.ds(base, CB)],
                    rr0.at[pl.ds(base_r, CB)],
                    rr1.at[pl.ds(base_r, CB)],
                    rr2.at[pl.ds(base_r, CB)],
                    rhr.at[pl.ds(base_r, CB)])

        def issue_loads(ck, bufs, sem):
            for sref, dref in zip(in_slices(ck), bufs):
                pltpu.async_copy(sref, dref, sem)

        def wait_loads(ck, bufs, sem):
            for sref, dref in zip(in_slices(ck), bufs):
                pltpu.make_async_copy(sref, dref, sem).wait()

        def drain_sc(nds):
            for _ in range(nds):
                pltpu.make_async_copy(mbuf, t_sh.at[dst_a], sem_sc).wait()

        def do_chunk(bufs):
            src_v, dst_v, r0_v, r1_v, r2_v, rh_v = bufs

            @pl.loop(0, CB // L)
            def _(g):
                b = g * L
                sidx = src_v[pl.ds(b, L)]
                sval = plsc.load_gather(s_v, [sidx])
                r0 = r0_v[pl.ds(b, L)]
                r1 = r1_v[pl.ds(b, L)]
                r2 = r2_v[pl.ds(b, L)]
                rh = rh_v[pl.ds(b, L)]
                mbuf[pl.ds(b, L)] = (
                    sval + r0 * p0 + r1 * p1 + r2 * p2 + rh * p3 + p4)
            pltpu.async_copy(mbuf, t_sh.at[dst_v], sem_sc, add=True)
            drain_sc(1)
            if first:
                ones = jnp.ones((L,), jnp.float32)

                @pl.loop(0, CB // L)
                def _(g):
                    mbuf[pl.ds(g * L, L)] = ones
                pltpu.async_copy(mbuf, deg_sh.at[dst_v], sem_sc, add=True)
                drain_sc(1)

        bufs_a = (src_a, dst_a, r0_a, r1_a, r2_a, rh_a)
        bufs_b = (src_b, dst_b, r0_b, r1_b, r2_b, rh_b)

        issue_loads(0, bufs_a, sem_a)

        @pl.loop(0, NCH // 2)
        def _(half):
            k = half * 2
            issue_loads(k + 1, bufs_b, sem_b)
            wait_loads(k, bufs_a, sem_a)
            do_chunk(bufs_a)

            @pl.when(k + 2 < NCH)
            def _():
                issue_loads(k + 2, bufs_a, sem_a)
            wait_loads(k + 1, bufs_b, sem_b)
            do_chunk(bufs_b)

        plsc.subcore_barrier()

        sl = pl.ds(base_n, TSL)
        osl = pl.ds(c * NP + base_n, TSL)
        pltpu.sync_copy(t_sh.at[sl], t_out.at[osl])

        if first:
            pltpu.sync_copy(deg_sh.at[sl], deg_out.at[osl])

    return pl.kernel(body, out_type=tuple(outs), mesh=_mesh,
                     scratch_types=scratch,
                     compiler_params=pltpu.CompilerParams(
                         needs_layout_passes=False))


def _finalize():
    scratch = [
        pltpu.VMEM((4 * UBLK,), jnp.float32),
        pltpu.VMEM((UBLK,), jnp.float32),
        pltpu.VMEM((UBLK,), jnp.float32),
        pltpu.VMEM((UBLK,), jnp.float32),
        pltpu.VMEM((UBLK,), jnp.float32),
        pltpu.VMEM((UBLK,), jnp.float32),
        pltpu.VMEM((UBLK,), jnp.float32),
        pltpu.VMEM((4 * UBLK,), jnp.float32),
        pltpu.VMEM((15, L), jnp.float32),
    ]

    def body(zf, a_in, t_in, deg_in, pp, x_out,
             zb, ab, t0b, t1b, g0b, g1b, a3b, xb, pv):
        c = lax.axis_index("c")
        s = lax.axis_index("s")
        wid = c * NS + s
        it = _iota16()
        pltpu.sync_copy(pp, pv)
        wd0, wd1, wd2, wd3, sdum = (pv[10], pv[11], pv[12], pv[13], pv[14])
        base_n = wid * (NP // NW)

        @pl.loop(0, NP // NW // UBLK)
        def _(blk):
            off = base_n + blk * UBLK
            pltpu.sync_copy(zf.at[pl.ds(4 * off, 4 * UBLK)], zb)
            pltpu.sync_copy(a_in.at[pl.ds(off, UBLK)], ab)
            pltpu.sync_copy(t_in.at[pl.ds(off, UBLK)], t0b)
            pltpu.sync_copy(t_in.at[pl.ds(NP + off, UBLK)], t1b)
            pltpu.sync_copy(deg_in.at[pl.ds(off, UBLK)], g0b)
            pltpu.sync_copy(deg_in.at[pl.ds(NP + off, UBLK)], g1b)
            @pl.loop(0, UBLK // L)
            def _(g):
                o = g * L
                zi = (it + o) * 4
                zv0 = plsc.load_gather(zb, [zi])
                zv1 = plsc.load_gather(zb, [zi + 1])
                zv2 = plsc.load_gather(zb, [zi + 2])
                zv3 = plsc.load_gather(zb, [zi + 3])
                av = ab[pl.ds(o, L)]
                dv = (zv0 * wd0 + zv1 * wd1 + zv2 * wd2 + zv3 * wd3
                      + sdum * av)
                a3b[pl.ds(o, L)] = (av + t0b[pl.ds(o, L)] + t1b[pl.ds(o, L)]
                                    + (g0b[pl.ds(o, L)] + g1b[pl.ds(o, L)])
                                    * dv)
            @pl.loop(0, UBLK // L)
            def _(g):
                for q in range(4):
                    p = g * 64 + q * L
                    ai = g * L + (q * 4) + (it >> 2)
                    av = plsc.load_gather(a3b, [ai])
                    xb[pl.ds(p, L)] = zb[pl.ds(p, L)] + av
            pltpu.sync_copy(xb, x_out.at[pl.ds(4 * off, 4 * UBLK)])

    return pl.kernel(
        body, out_type=jax.ShapeDtypeStruct((NP * 4,), jnp.float32),
        mesh=_mesh, scratch_types=scratch,
        compiler_params=pltpu.CompilerParams(needs_layout_passes=False))


def kernel(z, src, dst, r, r_hat, W, b):
    f32 = jnp.float32
    pad = EP - E
    pidx = jnp.arange(pad, dtype=jnp.int32) % 2048
    srcp = jnp.concatenate([src.astype(jnp.int32), pidx])
    dstp = jnp.concatenate([dst.astype(jnp.int32), pidx + N])
    rc0 = r[:, 0].astype(f32)
    rc1 = r[:, 1].astype(f32)
    rc2 = r[:, 2].astype(f32)
    rhf = r_hat[:, 0].astype(f32)
    zT = jnp.zeros((4, NP), f32).at[:, :N].set(z.astype(f32).T).reshape(-1)
    zf = jnp.zeros((NP, 4), f32).at[:N].set(z.astype(f32)).reshape(-1)

    Wf = W.astype(f32)
    bf = b.astype(f32)

    def params_for(i, iprev):
        rows = [Wf[i, 8, 0], Wf[i, 9, 0], Wf[i, 10, 0], Wf[i, 11, 0],
                bf[i, 0],
                Wf[i, 0, 0], Wf[i, 1, 0], Wf[i, 2, 0], Wf[i, 3, 0],
                jnp.sum(Wf[i, 0:4, 0])]
        if iprev is None:
            rows += [jnp.zeros(()), jnp.zeros(()), jnp.zeros(()),
                     jnp.zeros(()), jnp.zeros(())]
        else:
            rows += [Wf[iprev, 4, 0], Wf[iprev, 5, 0], Wf[iprev, 6, 0],
                     Wf[iprev, 7, 0], jnp.sum(Wf[iprev, 4:8, 0])]
        return jnp.broadcast_to(jnp.stack(rows)[:, None], (15, L))

    e_first = _edge_pass(True)
    e_next = _edge_pass(False)
    fin = _finalize()

    t0p, a0, _sb0, degp = e_first(zT, srcp, dstp, rc0, rc1, rc2, rhf,
                                  params_for(0, None))
    t1p, a1, _sb1 = e_next(zT, a0, t0p, degp, srcp, dstp, rc0, rc1, rc2,
                           rhf, params_for(1, 0))
    t2p, a2, _sb2 = e_next(zT, a1, t1p, degp, srcp, dstp, rc0, rc1, rc2,
                           rhf, params_for(2, 1))
    xf = fin(zf, a2, t2p, degp, params_for(0, 2))
    return xf.reshape(NP, 4)[:N]
